# Initial kernel scaffold; baseline (speedup 1.0000x reference)
import jax
import jax.numpy as jnp
from jax.experimental import pallas as pl


def kernel(x, W1, b1, g1, be1, W2, b2, g2, be2, W3, b3, g3, be3, fw1, fb1, fw2, fb2, fw3, fb3, fw4, fb4, edge_index):
    raise NotImplementedError("write your pallas kernel here")


# speedup vs baseline: 17.2316x; 17.2316x over previous
import functools

import jax
import jax.numpy as jnp
from jax import lax
from jax.experimental import pallas as pl
from jax.experimental.pallas import tpu as pltpu
from jax.experimental.pallas import tpu_sc as plsc

N_NODES = 10000
N_EDGES = 320000
ACC_ROWS = 10240
D = 128
GRP = 128
N_TILES = 32
K_PER_TILE = 80
N_GROUPS = K_PER_TILE * N_TILES
E_PAD = N_GROUPS * GRP
RPT = ACC_ROWS // 16

_MESH = plsc.VectorSubcoreMesh(core_axis_name="c", subcore_axis_name="s")


@functools.partial(
    pl.kernel, mesh=_MESH,
    out_type=(jax.ShapeDtypeStruct((ACC_ROWS, D), jnp.float32),
              jax.ShapeDtypeStruct((ACC_ROWS, D), jnp.float32)),
    scratch_types=[
        pltpu.VMEM((K_PER_TILE, GRP), jnp.int32),
        pltpu.VMEM((K_PER_TILE, GRP), jnp.int32),
        pltpu.VMEM((GRP, D), jnp.float32),
        pltpu.VMEM_SHARED((ACC_ROWS, D), jnp.float32),
        pltpu.SemaphoreType.DMA,
    ])
def _agg(u_hbm, z_hbm, srcg_hbm, dstg_hbm, p0_hbm, p1_hbm,
         srcv, dstv, rows, acc, sem):
  c = lax.axis_index("c")
  s = lax.axis_index("s")
  tile = c * 16 + s

  @pl.when(c == 0)
  def _():
    pltpu.sync_copy(u_hbm.at[pl.ds(s * RPT, RPT)],
                    acc.at[pl.ds(s * RPT, RPT)])

  @pl.when(c == 1)
  def _():
    pltpu.sync_copy(z_hbm.at[pl.ds(s * RPT, RPT)],
                    acc.at[pl.ds(s * RPT, RPT)])

  pltpu.sync_copy(srcg_hbm.at[pl.ds(tile * K_PER_TILE, K_PER_TILE)], srcv)
  pltpu.sync_copy(dstg_hbm.at[pl.ds(tile * K_PER_TILE, K_PER_TILE)], dstv)
  plsc.subcore_barrier()

  def body(j, carry):
    pltpu.async_copy(u_hbm.at[srcv.at[j]], rows, sem).wait()
    pltpu.sync_copy(rows, acc.at[dstv.at[j]], add=True)
    return carry

  lax.fori_loop(0, K_PER_TILE, body, 0)
  plsc.subcore_barrier()

  @pl.when(c == 0)
  def _():
    pltpu.sync_copy(acc.at[pl.ds(s * RPT, RPT)],
                    p0_hbm.at[pl.ds(s * RPT, RPT)])

  @pl.when(c == 1)
  def _():
    pltpu.sync_copy(acc.at[pl.ds(s * RPT, RPT)],
                    p1_hbm.at[pl.ds(s * RPT, RPT)])


@functools.partial(
    pl.kernel, mesh=_MESH,
    out_type=(jax.ShapeDtypeStruct((ACC_ROWS, D), jnp.float32),
              jax.ShapeDtypeStruct((ACC_ROWS, D), jnp.float32)),
    scratch_types=[
        pltpu.VMEM((K_PER_TILE, GRP), jnp.int32),
        pltpu.VMEM((GRP, D), jnp.float32),
        pltpu.VMEM_SHARED((ACC_ROWS, D), jnp.float32),
    ])
def _deg(ones_hbm, z_hbm, dstg_hbm, p0_hbm, p1_hbm, dstv, onesv, acc):
  c = lax.axis_index("c")
  s = lax.axis_index("s")
  tile = c * 16 + s

  @pl.when(c == 0)
  def _():
    pltpu.sync_copy(ones_hbm.at[pl.ds(s * RPT, RPT)],
                    acc.at[pl.ds(s * RPT, RPT)])

  @pl.when(c == 1)
  def _():
    pltpu.sync_copy(z_hbm.at[pl.ds(s * RPT, RPT)],
                    acc.at[pl.ds(s * RPT, RPT)])

  pltpu.sync_copy(ones_hbm.at[pl.ds(0, GRP)], onesv)
  pltpu.sync_copy(dstg_hbm.at[pl.ds(tile * K_PER_TILE, K_PER_TILE)], dstv)
  plsc.subcore_barrier()

  def body(j, carry):
    pltpu.sync_copy(onesv, acc.at[dstv.at[j]], add=True)
    return carry

  lax.fori_loop(0, K_PER_TILE, body, 0)
  plsc.subcore_barrier()

  @pl.when(c == 0)
  def _():
    pltpu.sync_copy(acc.at[pl.ds(s * RPT, RPT)],
                    p0_hbm.at[pl.ds(s * RPT, RPT)])

  @pl.when(c == 1)
  def _():
    pltpu.sync_copy(acc.at[pl.ds(s * RPT, RPT)],
                    p1_hbm.at[pl.ds(s * RPT, RPT)])


def _tc1(deg0_ref, deg1_ref, x_ref, w1_ref, dinv_ref, u1_ref):
  deg = deg0_ref[:, 0:1] + deg1_ref[:, 0:1]
  dinv = lax.rsqrt(deg)
  dinv_ref[...] = dinv
  h = jnp.dot(x_ref[...], w1_ref[...], preferred_element_type=jnp.float32)
  u1_ref[...] = h * dinv


def _bn(t):
  mask = lax.broadcasted_iota(jnp.int32, (ACC_ROWS, 1), 0) < N_NODES
  tm = jnp.where(mask, t, 0.0)
  mean = jnp.sum(tm, axis=0, keepdims=True) * (1.0 / N_NODES)
  cen = t - mean
  var = jnp.sum(jnp.where(mask, cen * cen, 0.0), axis=0,
                keepdims=True) * (1.0 / N_NODES)
  return cen * lax.rsqrt(var + 1e-5)


def _tc_mid(p0_ref, p1_ref, dinv_ref, b_ref, g_ref, be_ref, w_ref, u_ref):
  dv = dinv_ref[...]
  t = (p0_ref[...] + p1_ref[...]) * dv + b_ref[...]
  y = jnp.maximum(_bn(t) * g_ref[...] + be_ref[...], 0.0)
  h = jnp.dot(y, w_ref[...], preferred_element_type=jnp.float32)
  u_ref[...] = h * dv


def _tc_head(p0_ref, p1_ref, dinv_ref, b_ref, g_ref, be_ref,
             fw1_ref, fb1_ref, fw2_ref, fb2_ref, fw3_ref, fb3_ref,
             fw4_ref, fb4_ref, out_ref):
  t = (p0_ref[...] + p1_ref[...]) * dinv_ref[...] + b_ref[...]
  h = jnp.maximum(_bn(t) * g_ref[...] + be_ref[...], 0.0)
  h = jnp.maximum(jnp.dot(h, fw1_ref[...],
                          preferred_element_type=jnp.float32) + fb1_ref[...],
                  0.0)
  h = jnp.maximum(jnp.dot(h, fw2_ref[...],
                          preferred_element_type=jnp.float32) + fb2_ref[...],
                  0.0)
  h = jnp.maximum(jnp.dot(h, fw3_ref[...],
                          preferred_element_type=jnp.float32) + fb3_ref[...],
                  0.0)
  out_ref[...] = jnp.dot(h, fw4_ref[...],
                         preferred_element_type=jnp.float32) + fb4_ref[...]


def _colpad(w, n):
  return jnp.pad(w, ((0, 0), (0, n - w.shape[1])))


def kernel(x, W1, b1, g1, be1, W2, b2, g2, be2, W3, b3, g3, be3,
           fw1, fb1, fw2, fb2, fw3, fb3, fw4, fb4, edge_index):
  f32 = jnp.float32

  pad = E_PAD - N_EDGES
  ar = jnp.arange(pad, dtype=jnp.int32)
  pad_src = ar % N_NODES
  pad_dst = N_NODES + ar % (ACC_ROWS - N_NODES)
  src_g = jnp.concatenate([edge_index[0], pad_src]).reshape(N_GROUPS, GRP)
  dst_g = jnp.concatenate([edge_index[1], pad_dst]).reshape(N_GROUPS, GRP)

  x_pad = jnp.pad(x, ((0, ACC_ROWS - N_NODES), (0, 0)))
  ones128 = jnp.ones((ACC_ROWS, D), f32)
  z128 = jnp.zeros((ACC_ROWS, D), f32)

  w1p = _colpad(W1, D)
  w2p = jnp.pad(W2, ((0, D - 32), (0, D - 64)))
  w3p = jnp.pad(W3, ((0, D - 64), (0, 0)))
  b1p = _colpad(b1.reshape(1, -1), D)
  g1p = _colpad(g1.reshape(1, -1), D)
  be1p = _colpad(be1.reshape(1, -1), D)
  b2p = _colpad(b2.reshape(1, -1), D)
  g2p = _colpad(g2.reshape(1, -1), D)
  be2p = _colpad(be2.reshape(1, -1), D)

  deg0, deg1 = _deg(ones128, z128, dst_g)

  dinv, u1 = pl.pallas_call(
      _tc1,
      out_shape=(jax.ShapeDtypeStruct((ACC_ROWS, 1), f32),
                 jax.ShapeDtypeStruct((ACC_ROWS, D), f32)),
  )(deg0, deg1, x_pad, w1p)

  a0, a1 = _agg(u1, z128, src_g, dst_g)
  u2 = pl.pallas_call(
      _tc_mid,
      out_shape=jax.ShapeDtypeStruct((ACC_ROWS, D), f32),
  )(a0, a1, dinv, b1p, g1p, be1p, w2p)

  b0, b1q = _agg(u2, z128, src_g, dst_g)
  u3 = pl.pallas_call(
      _tc_mid,
      out_shape=jax.ShapeDtypeStruct((ACC_ROWS, D), f32),
  )(b0, b1q, dinv, b2p, g2p, be2p, w3p)

  c0, c1 = _agg(u3, z128, src_g, dst_g)
  out = pl.pallas_call(
      _tc_head,
      out_shape=jax.ShapeDtypeStruct((ACC_ROWS, 40), f32),
  )(c0, c1, dinv, b3.reshape(1, -1), g3.reshape(1, -1), be3.reshape(1, -1),
    fw1, fb1.reshape(1, -1), fw2, fb2.reshape(1, -1),
    fw3, fb3.reshape(1, -1), fw4, fb4.reshape(1, -1))

  return out[:N_NODES]


# speedup vs baseline: 22.4097x; 1.3005x over previous
import functools

import jax
import jax.numpy as jnp
from jax import lax
from jax.experimental import pallas as pl
from jax.experimental.pallas import tpu as pltpu
from jax.experimental.pallas import tpu_sc as plsc

N_NODES = 10000
N_EDGES = 320000
ACC_ROWS = 10240
D = 128
GRP = 128
N_TILES = 32
K_PER_TILE = 80
NBUF = 4
N_GROUPS = K_PER_TILE * N_TILES
E_PAD = N_GROUPS * GRP
RPT = ACC_ROWS // 16

_MESH = plsc.VectorSubcoreMesh(core_axis_name="c", subcore_axis_name="s")


@functools.partial(
    pl.kernel, mesh=_MESH,
    out_type=(jax.ShapeDtypeStruct((ACC_ROWS, D), jnp.float32),
              jax.ShapeDtypeStruct((ACC_ROWS, D), jnp.float32)),
    scratch_types=[
        pltpu.VMEM((K_PER_TILE // 2, GRP), jnp.int32),
        pltpu.VMEM((K_PER_TILE // 2, GRP), jnp.int32),
        pltpu.VMEM((GRP, D), jnp.float32),
        pltpu.VMEM((GRP, D), jnp.float32),
        pltpu.VMEM_SHARED((ACC_ROWS, D), jnp.float32),
        pltpu.SemaphoreType.DMA,
        pltpu.SemaphoreType.DMA,
    ])
def _agg(u_hbm, z_hbm, srcg_hbm, dstg_hbm, p0_hbm, p1_hbm,
         srcv, dstv, rows0, rows1, acc, semg0, semg1):
  c = lax.axis_index("c")
  s = lax.axis_index("s")
  tile = c * 16 + s
  rows = (rows0, rows1)
  sems = (semg0, semg1)
  KH = K_PER_TILE // 2

  @pl.when(c == 0)
  def _():
    pltpu.sync_copy(u_hbm.at[pl.ds(s * RPT, RPT)],
                    acc.at[pl.ds(s * RPT, RPT)])

  @pl.when(c == 1)
  def _():
    pltpu.sync_copy(z_hbm.at[pl.ds(s * RPT, RPT)],
                    acc.at[pl.ds(s * RPT, RPT)])
  plsc.subcore_barrier()

  def gather(g, i):
    return pltpu.make_async_copy(u_hbm.at[srcv.at[g]], rows[i], sems[i])

  def scat(g, i):
    pltpu.sync_copy(rows[i], acc.at[dstv.at[g]], add=True)

  for h in range(2):
    pltpu.sync_copy(srcg_hbm.at[pl.ds(tile * K_PER_TILE + h * KH, KH)], srcv)
    pltpu.sync_copy(dstg_hbm.at[pl.ds(tile * K_PER_TILE + h * KH, KH)], dstv)
    gather(0, 0).start()

    def body(t, carry):
      g0 = 2 * t
      gather(g0 + 1, 1).start()
      gather(g0, 0).wait()
      scat(g0, 0)
      gather(g0 + 2, 0).start()
      gather(g0 + 1, 1).wait()
      scat(g0 + 1, 1)
      return carry

    lax.fori_loop(0, KH // 2 - 1, body, 0)
    g0 = KH - 2
    gather(g0 + 1, 1).start()
    gather(g0, 0).wait()
    scat(g0, 0)
    gather(g0 + 1, 1).wait()
    scat(g0 + 1, 1)

  plsc.subcore_barrier()

  @pl.when(c == 0)
  def _():
    pltpu.sync_copy(acc.at[pl.ds(s * RPT, RPT)],
                    p0_hbm.at[pl.ds(s * RPT, RPT)])

  @pl.when(c == 1)
  def _():
    pltpu.sync_copy(acc.at[pl.ds(s * RPT, RPT)],
                    p1_hbm.at[pl.ds(s * RPT, RPT)])


def _tc1(deg0_ref, deg1_ref, x_ref, w1_ref, dinv_ref, u1_ref):
  deg = deg0_ref[:, 0:1] + deg1_ref[:, 0:1]
  dinv = lax.rsqrt(deg)
  dinv_ref[...] = dinv
  h = jnp.dot(x_ref[...], w1_ref[...], preferred_element_type=jnp.float32)
  u1_ref[...] = h * dinv


def _bn(t):
  mask = lax.broadcasted_iota(jnp.int32, (ACC_ROWS, 1), 0) < N_NODES
  tm = jnp.where(mask, t, 0.0)
  mean = jnp.sum(tm, axis=0, keepdims=True) * (1.0 / N_NODES)
  cen = t - mean
  var = jnp.sum(jnp.where(mask, cen * cen, 0.0), axis=0,
                keepdims=True) * (1.0 / N_NODES)
  return cen * lax.rsqrt(var + 1e-5)


def _tc_mid(p0_ref, p1_ref, dinv_ref, b_ref, g_ref, be_ref, w_ref, u_ref):
  dv = dinv_ref[...]
  t = (p0_ref[...] + p1_ref[...]) * dv + b_ref[...]
  y = jnp.maximum(_bn(t) * g_ref[...] + be_ref[...], 0.0)
  h = jnp.dot(y, w_ref[...], preferred_element_type=jnp.float32)
  u_ref[...] = h * dv


def _tc_head(p0_ref, p1_ref, dinv_ref, b_ref, g_ref, be_ref,
             fw1_ref, fb1_ref, fw2_ref, fb2_ref, fw3_ref, fb3_ref,
             fw4_ref, fb4_ref, out_ref):
  t = (p0_ref[...] + p1_ref[...]) * dinv_ref[...] + b_ref[...]
  h = jnp.maximum(_bn(t) * g_ref[...] + be_ref[...], 0.0)
  h = jnp.maximum(jnp.dot(h, fw1_ref[...],
                          preferred_element_type=jnp.float32) + fb1_ref[...],
                  0.0)
  h = jnp.maximum(jnp.dot(h, fw2_ref[...],
                          preferred_element_type=jnp.float32) + fb2_ref[...],
                  0.0)
  h = jnp.maximum(jnp.dot(h, fw3_ref[...],
                          preferred_element_type=jnp.float32) + fb3_ref[...],
                  0.0)
  out_ref[...] = jnp.dot(h, fw4_ref[...],
                         preferred_element_type=jnp.float32) + fb4_ref[...]


def _colpad(w, n):
  return jnp.pad(w, ((0, 0), (0, n - w.shape[1])))


def kernel(x, W1, b1, g1, be1, W2, b2, g2, be2, W3, b3, g3, be3,
           fw1, fb1, fw2, fb2, fw3, fb3, fw4, fb4, edge_index):
  f32 = jnp.float32

  pad = E_PAD - N_EDGES
  ar = jnp.arange(pad, dtype=jnp.int32)
  pad_src = ar % N_NODES
  pad_dst = N_NODES + ar % (ACC_ROWS - N_NODES)
  src_g = jnp.concatenate([edge_index[0], pad_src]).reshape(-1, GRP)
  dst_g = jnp.concatenate([edge_index[1], pad_dst]).reshape(-1, GRP)

  x_pad = jnp.pad(x, ((0, ACC_ROWS - N_NODES), (0, 0)))
  ones128 = jnp.ones((ACC_ROWS, D), f32)
  z128 = jnp.zeros((ACC_ROWS, D), f32)

  w1p = _colpad(W1, D)
  w2p = jnp.pad(W2, ((0, D - 32), (0, D - 64)))
  w3p = jnp.pad(W3, ((0, D - 64), (0, 0)))
  b1p = _colpad(b1.reshape(1, -1), D)
  g1p = _colpad(g1.reshape(1, -1), D)
  be1p = _colpad(be1.reshape(1, -1), D)
  b2p = _colpad(b2.reshape(1, -1), D)
  g2p = _colpad(g2.reshape(1, -1), D)
  be2p = _colpad(be2.reshape(1, -1), D)

  deg0, deg1 = _agg(ones128, z128, src_g, dst_g)

  dinv, u1 = pl.pallas_call(
      _tc1,
      out_shape=(jax.ShapeDtypeStruct((ACC_ROWS, 1), f32),
                 jax.ShapeDtypeStruct((ACC_ROWS, D), f32)),
  )(deg0, deg1, x_pad, w1p)

  a0, a1 = _agg(u1, z128, src_g, dst_g)
  u2 = pl.pallas_call(
      _tc_mid,
      out_shape=jax.ShapeDtypeStruct((ACC_ROWS, D), f32),
  )(a0, a1, dinv, b1p, g1p, be1p, w2p)

  b0, b1q = _agg(u2, z128, src_g, dst_g)
  u3 = pl.pallas_call(
      _tc_mid,
      out_shape=jax.ShapeDtypeStruct((ACC_ROWS, D), f32),
  )(b0, b1q, dinv, b2p, g2p, be2p, w3p)

  c0, c1 = _agg(u3, z128, src_g, dst_g)
  out = pl.pallas_call(
      _tc_head,
      out_shape=jax.ShapeDtypeStruct((ACC_ROWS, 40), f32),
  )(c0, c1, dinv, b3.reshape(1, -1), g3.reshape(1, -1), be3.reshape(1, -1),
    fw1, fb1.reshape(1, -1), fw2, fb2.reshape(1, -1),
    fw3, fb3.reshape(1, -1), fw4, fb4.reshape(1, -1))

  return out[:N_NODES]


# speedup vs baseline: 29.7571x; 1.3279x over previous
import functools

import jax
import jax.numpy as jnp
from jax import lax
from jax.experimental import pallas as pl
from jax.experimental.pallas import tpu as pltpu
from jax.experimental.pallas import tpu_sc as plsc

N_NODES = 10000
N_EDGES = 320000
ACC_ROWS = 10240
D = 128
GRP = 128
N_TILES = 32
K_PER_TILE = 80
NBUF = 4
N_GROUPS = K_PER_TILE * N_TILES
E_PAD = N_GROUPS * GRP
RPT = ACC_ROWS // 16

_MESH = plsc.VectorSubcoreMesh(core_axis_name="c", subcore_axis_name="s")


def _make_agg(W):
  @functools.partial(
      pl.kernel, mesh=_MESH,
      out_type=(jax.ShapeDtypeStruct((ACC_ROWS, W), jnp.float32),
                jax.ShapeDtypeStruct((ACC_ROWS, W), jnp.float32)),
      scratch_types=[
          pltpu.VMEM((K_PER_TILE // 2, GRP), jnp.int32),
          pltpu.VMEM((K_PER_TILE // 2, GRP), jnp.int32),
          pltpu.VMEM((GRP, W), jnp.float32),
          pltpu.VMEM((GRP, W), jnp.float32),
          pltpu.VMEM_SHARED((ACC_ROWS, W), jnp.float32),
          pltpu.SemaphoreType.DMA,
          pltpu.SemaphoreType.DMA,
      ],
      compiler_params=pltpu.CompilerParams(use_tc_tiling_on_sc=False))
  def _agg(u_hbm, z_hbm, srcg_hbm, dstg_hbm, p0_hbm, p1_hbm,
           srcv, dstv, rows0, rows1, acc, semg0, semg1):
    c = lax.axis_index("c")
    s = lax.axis_index("s")
    tile = c * 16 + s
    rows = (rows0, rows1)
    sems = (semg0, semg1)
    KH = K_PER_TILE // 2

    @pl.when(c == 0)
    def _():
      pltpu.sync_copy(u_hbm.at[pl.ds(s * RPT, RPT)],
                      acc.at[pl.ds(s * RPT, RPT)])

    @pl.when(c == 1)
    def _():
      pltpu.sync_copy(z_hbm.at[pl.ds(s * RPT, RPT)],
                      acc.at[pl.ds(s * RPT, RPT)])
    plsc.subcore_barrier()

    def gather(g, i):
      return pltpu.make_async_copy(u_hbm.at[srcv.at[g]], rows[i], sems[i])

    def scat(g, i):
      pltpu.sync_copy(rows[i], acc.at[dstv.at[g]], add=True)

    for h in range(2):
      pltpu.sync_copy(srcg_hbm.at[pl.ds(tile * K_PER_TILE + h * KH, KH)],
                      srcv)
      pltpu.sync_copy(dstg_hbm.at[pl.ds(tile * K_PER_TILE + h * KH, KH)],
                      dstv)
      gather(0, 0).start()

      def body(t, carry):
        g0 = 2 * t
        gather(g0 + 1, 1).start()
        gather(g0, 0).wait()
        scat(g0, 0)
        gather(g0 + 2, 0).start()
        gather(g0 + 1, 1).wait()
        scat(g0 + 1, 1)
        return carry

      lax.fori_loop(0, KH // 2 - 1, body, 0)
      g0 = KH - 2
      gather(g0 + 1, 1).start()
      gather(g0, 0).wait()
      scat(g0, 0)
      gather(g0 + 1, 1).wait()
      scat(g0 + 1, 1)

    plsc.subcore_barrier()

    @pl.when(c == 0)
    def _():
      pltpu.sync_copy(acc.at[pl.ds(s * RPT, RPT)],
                      p0_hbm.at[pl.ds(s * RPT, RPT)])

    @pl.when(c == 1)
    def _():
      pltpu.sync_copy(acc.at[pl.ds(s * RPT, RPT)],
                      p1_hbm.at[pl.ds(s * RPT, RPT)])

  return _agg


def _tc1(deg0_ref, deg1_ref, x_ref, w1_ref, dinv_ref, u1_ref):
  deg = deg0_ref[:, 0:1] + deg1_ref[:, 0:1]
  dinv = lax.rsqrt(deg)
  dinv_ref[...] = dinv
  h = jnp.dot(x_ref[...], w1_ref[...], preferred_element_type=jnp.float32)
  u1_ref[...] = h * dinv


def _bn(t):
  mask = lax.broadcasted_iota(jnp.int32, (ACC_ROWS, 1), 0) < N_NODES
  tm = jnp.where(mask, t, 0.0)
  mean = jnp.sum(tm, axis=0, keepdims=True) * (1.0 / N_NODES)
  cen = t - mean
  var = jnp.sum(jnp.where(mask, cen * cen, 0.0), axis=0,
                keepdims=True) * (1.0 / N_NODES)
  return cen * lax.rsqrt(var + 1e-5)


def _tc_mid(p0_ref, p1_ref, dinv_ref, b_ref, g_ref, be_ref, w_ref, u_ref):
  dv = dinv_ref[...]
  t = (p0_ref[...] + p1_ref[...]) * dv + b_ref[...]
  y = jnp.maximum(_bn(t) * g_ref[...] + be_ref[...], 0.0)
  h = jnp.dot(y, w_ref[...], preferred_element_type=jnp.float32)
  u_ref[...] = h * dv


def _tc_head(p0_ref, p1_ref, dinv_ref, b_ref, g_ref, be_ref,
             fw1_ref, fb1_ref, fw2_ref, fb2_ref, fw3_ref, fb3_ref,
             fw4_ref, fb4_ref, out_ref):
  t = (p0_ref[...] + p1_ref[...]) * dinv_ref[...] + b_ref[...]
  h = jnp.maximum(_bn(t) * g_ref[...] + be_ref[...], 0.0)
  h = jnp.maximum(jnp.dot(h, fw1_ref[...],
                          preferred_element_type=jnp.float32) + fb1_ref[...],
                  0.0)
  h = jnp.maximum(jnp.dot(h, fw2_ref[...],
                          preferred_element_type=jnp.float32) + fb2_ref[...],
                  0.0)
  h = jnp.maximum(jnp.dot(h, fw3_ref[...],
                          preferred_element_type=jnp.float32) + fb3_ref[...],
                  0.0)
  out_ref[...] = jnp.dot(h, fw4_ref[...],
                         preferred_element_type=jnp.float32) + fb4_ref[...]


def _colpad(w, n):
  return jnp.pad(w, ((0, 0), (0, n - w.shape[1])))


def kernel(x, W1, b1, g1, be1, W2, b2, g2, be2, W3, b3, g3, be3,
           fw1, fb1, fw2, fb2, fw3, fb3, fw4, fb4, edge_index):
  f32 = jnp.float32

  pad = E_PAD - N_EDGES
  ar = jnp.arange(pad, dtype=jnp.int32)
  pad_src = ar % N_NODES
  pad_dst = N_NODES + ar % (ACC_ROWS - N_NODES)
  src_g = jnp.concatenate([edge_index[0], pad_src]).reshape(-1, GRP)
  dst_g = jnp.concatenate([edge_index[1], pad_dst]).reshape(-1, GRP)

  x_pad = jnp.pad(x, ((0, ACC_ROWS - N_NODES), (0, 0)))
  ones16 = jnp.ones((ACC_ROWS, 16), f32)
  z16 = jnp.zeros((ACC_ROWS, 16), f32)
  z32 = jnp.zeros((ACC_ROWS, 32), f32)
  z64 = jnp.zeros((ACC_ROWS, 64), f32)
  z128 = jnp.zeros((ACC_ROWS, D), f32)


  deg0, deg1 = _make_agg(16)(ones16, z16, src_g, dst_g)

  dinv, u1 = pl.pallas_call(
      _tc1,
      out_shape=(jax.ShapeDtypeStruct((ACC_ROWS, 1), f32),
                 jax.ShapeDtypeStruct((ACC_ROWS, 32), f32)),
  )(deg0, deg1, x_pad, W1)

  a0, a1 = _make_agg(32)(u1, z32, src_g, dst_g)
  u2 = pl.pallas_call(
      _tc_mid,
      out_shape=jax.ShapeDtypeStruct((ACC_ROWS, 64), f32),
  )(a0, a1, dinv, b1.reshape(1, -1), g1.reshape(1, -1), be1.reshape(1, -1),
    W2)

  b0, b1q = _make_agg(64)(u2, z64, src_g, dst_g)
  u3 = pl.pallas_call(
      _tc_mid,
      out_shape=jax.ShapeDtypeStruct((ACC_ROWS, 128), f32),
  )(b0, b1q, dinv, b2.reshape(1, -1), g2.reshape(1, -1), be2.reshape(1, -1),
    W3)

  c0, c1 = _make_agg(128)(u3, z128, src_g, dst_g)
  out = pl.pallas_call(
      _tc_head,
      out_shape=jax.ShapeDtypeStruct((ACC_ROWS, 40), f32),
  )(c0, c1, dinv, b3.reshape(1, -1), g3.reshape(1, -1), be3.reshape(1, -1),
    fw1, fb1.reshape(1, -1), fw2, fb2.reshape(1, -1),
    fw3, fb3.reshape(1, -1), fw4, fb4.reshape(1, -1))

  return out[:N_NODES]
